# speedup vs baseline: 13.8910x; 13.8910x over previous
import jax
import jax.numpy as jnp
from jax import lax
from jax.experimental import pallas as pl
from jax.experimental.pallas import tpu as pltpu
from jax.experimental.pallas import tpu_sc as plsc

NC = 2
NS = 16
NW = NC * NS
PB = 128


def _deg_kernel(npad, nb):
    mesh = plsc.VectorSubcoreMesh(core_axis_name="c", subcore_axis_name="s")
    rpt = npad // NS

    def body(cols_hbm, deg_out, idx_v, ones_v, zero_v, deg_sh):
        c = lax.axis_index("c")
        s = lax.axis_index("s")
        wid = c * NS + s
        for i in range(PB // 16):
            ones_v[pl.ds(i * 16, 16)] = jnp.ones((16,), jnp.float32)
        for i in range(rpt // 16):
            zero_v[pl.ds(i * 16, 16)] = jnp.zeros((16,), jnp.float32)
        pltpu.sync_copy(zero_v, deg_sh.at[pl.ds(s * rpt, rpt)])
        pltpu.sync_copy(cols_hbm.at[pl.ds(wid * nb, nb)], idx_v)
        plsc.subcore_barrier()

        def blk(j, carry):
            pltpu.sync_copy(ones_v, deg_sh.at[idx_v.at[j]], add=True)
            return carry

        lax.fori_loop(0, nb, blk, 0)
        plsc.subcore_barrier()
        pltpu.sync_copy(deg_sh.at[pl.ds(s * rpt, rpt)],
                        deg_out.at[pl.ds(c * npad + s * rpt, rpt)])

    return pl.kernel(
        body,
        out_type=jax.ShapeDtypeStruct((NC * npad,), jnp.float32),
        mesh=mesh,
        scratch_types=[
            pltpu.VMEM((nb, PB), jnp.int32),
            pltpu.VMEM((PB,), jnp.float32),
            pltpu.VMEM((rpt,), jnp.float32),
            pltpu.VMEM_SHARED((npad,), jnp.float32),
        ],
    )


def _scatter_kernel(npad, nb, d):
    mesh = plsc.VectorSubcoreMesh(core_axis_name="c", subcore_axis_name="s")
    rpt = npad // NS
    nb2 = nb // 2

    def body(rows_hbm, cols_hbm, y_hbm, zinit_hbm, out_hbm,
             idxr_v, idxc_v, msg_v, acc_sh, sem):
        c = lax.axis_index("c")
        s = lax.axis_index("s")
        wid = c * NS + s
        base_blk = wid * nb
        pltpu.sync_copy(zinit_hbm.at[pl.ds(s * rpt, rpt)],
                        acc_sh.at[pl.ds(s * rpt, rpt)])
        plsc.subcore_barrier()
        for b in range(2):
            pltpu.sync_copy(rows_hbm.at[base_blk + b], idxr_v.at[b])
            pltpu.sync_copy(cols_hbm.at[base_blk + b], idxc_v.at[b])
            pltpu.async_copy(y_hbm.at[idxr_v.at[b]], msg_v.at[b], sem)

        def pair(k, carry):
            for b in range(2):
                j = 2 * k + b
                pltpu.make_async_copy(y_hbm.at[idxr_v.at[b]], msg_v.at[b],
                                      sem).wait()
                pltpu.sync_copy(msg_v.at[b], acc_sh.at[idxc_v.at[b]],
                                add=True)

                @pl.when(k < nb2 - 1)
                def _():
                    pltpu.sync_copy(rows_hbm.at[base_blk + j + 2],
                                    idxr_v.at[b])
                    pltpu.sync_copy(cols_hbm.at[base_blk + j + 2],
                                    idxc_v.at[b])
                    pltpu.async_copy(y_hbm.at[idxr_v.at[b]], msg_v.at[b], sem)
            return carry

        lax.fori_loop(0, nb2, pair, 0)
        plsc.subcore_barrier()
        pltpu.sync_copy(acc_sh.at[pl.ds(s * rpt, rpt)],
                        out_hbm.at[pl.ds(c * npad + s * rpt, rpt)])

    return pl.kernel(
        body,
        out_type=jax.ShapeDtypeStruct((NC * npad, d), jnp.float32),
        mesh=mesh,
        scratch_types=[
            pltpu.VMEM((2, PB), jnp.int32),
            pltpu.VMEM((2, PB), jnp.int32),
            pltpu.VMEM((2, PB, d), jnp.float32),
            pltpu.VMEM_SHARED((npad, d), jnp.float32),
            pltpu.SemaphoreType.DMA,
        ],
    )


def _transform_kernel(npad, d_in, d_out, br):
    def body(x_ref, wt_ref, d0_ref, d1_ref, y_ref):
        deg = d0_ref[...] + d1_ref[...] + 1.0
        dinv = lax.rsqrt(deg)
        xw = jnp.dot(x_ref[...], wt_ref[...],
                     preferred_element_type=jnp.float32)
        y_ref[...] = xw * dinv

    return pl.pallas_call(
        body,
        grid=(npad // br,),
        in_specs=[
            pl.BlockSpec((br, d_in), lambda i: (i, 0)),
            pl.BlockSpec((d_in, d_out), lambda i: (0, 0)),
            pl.BlockSpec((br, 1), lambda i: (i, 0)),
            pl.BlockSpec((br, 1), lambda i: (i, 0)),
        ],
        out_specs=pl.BlockSpec((br, d_out), lambda i: (i, 0)),
        out_shape=jax.ShapeDtypeStruct((npad, d_out), jnp.float32),
    )


def _final_kernel(n, d, br):
    def body(p0_ref, p1_ref, y_ref, d0_ref, d1_ref, o_ref):
        deg = d0_ref[...] + d1_ref[...] + 1.0
        dinv = lax.rsqrt(deg)
        o_ref[...] = (p0_ref[...] + p1_ref[...] + y_ref[...]) * dinv

    return pl.pallas_call(
        body,
        grid=(n // br,),
        in_specs=[
            pl.BlockSpec((br, d), lambda i: (i, 0)),
            pl.BlockSpec((br, d), lambda i: (i, 0)),
            pl.BlockSpec((br, d), lambda i: (i, 0)),
            pl.BlockSpec((br, 1), lambda i: (i, 0)),
            pl.BlockSpec((br, 1), lambda i: (i, 0)),
        ],
        out_specs=pl.BlockSpec((br, d), lambda i: (i, 0)),
        out_shape=jax.ShapeDtypeStruct((n, d), jnp.float32),
    )


def kernel(x, edge_index, num_nodes, W):
    n, d_in = x.shape
    d_out = W.shape[0]
    e = edge_index.shape[1]
    del num_nodes

    npad = -(-n // (NS * 16)) * (NS * 16)
    nb = -(-e // (NW * PB))
    nb += nb % 2
    ep = NW * nb * PB

    row = edge_index[0]
    col = edge_index[1]
    padn = ep - e
    rows2 = jnp.concatenate(
        [row, jnp.zeros((padn,), edge_index.dtype)]).reshape(NW * nb, PB)
    cols2 = jnp.concatenate(
        [col, jnp.full((padn,), npad - 1, edge_index.dtype)]
    ).reshape(NW * nb, PB)
    xp = jnp.concatenate([x, jnp.zeros((npad - n, d_in), x.dtype)])
    wt = W.T
    zinit = jnp.zeros((npad, d_out), jnp.float32)

    degp = _deg_kernel(npad, nb)(cols2)
    d0c = degp[:npad].reshape(npad, 1)
    d1c = degp[npad:].reshape(npad, 1)

    y = _transform_kernel(npad, d_in, d_out, 1024)(xp, wt, d0c, d1c)

    accp = _scatter_kernel(npad, nb, d_out)(rows2, cols2, y, zinit)

    out = _final_kernel(n, d_out, 1000)(
        accp[:n], accp[npad:npad + n], y[:n], d0c[:n], d1c[:n])
    return out


# speedup vs baseline: 33.3552x; 2.4012x over previous
import jax
import jax.numpy as jnp
from jax import lax
from jax.experimental import pallas as pl
from jax.experimental.pallas import tpu as pltpu
from jax.experimental.pallas import tpu_sc as plsc

NC = 2
NS = 16
NW = NC * NS
PB = 128


def _deg_kernel(npad, nb):
    mesh = plsc.VectorSubcoreMesh(core_axis_name="c", subcore_axis_name="s")
    rpt = npad // NS

    def body(cols_hbm, deg_out, idx_v, ones_v, zero_v, deg_sh):
        c = lax.axis_index("c")
        s = lax.axis_index("s")
        wid = c * NS + s
        for i in range(PB // 16):
            ones_v[pl.ds(i * 16, 16)] = jnp.ones((16,), jnp.float32)
        for i in range(rpt // 16):
            zero_v[pl.ds(i * 16, 16)] = jnp.zeros((16,), jnp.float32)
        pltpu.sync_copy(zero_v, deg_sh.at[pl.ds(s * rpt, rpt)])
        pltpu.sync_copy(cols_hbm.at[pl.ds(wid * nb, nb)], idx_v)
        plsc.subcore_barrier()

        def blk(j, carry):
            pltpu.sync_copy(ones_v, deg_sh.at[idx_v.at[j]], add=True)
            return carry

        lax.fori_loop(0, nb, blk, 0)
        plsc.subcore_barrier()
        pltpu.sync_copy(deg_sh.at[pl.ds(s * rpt, rpt)],
                        deg_out.at[pl.ds(c * npad + s * rpt, rpt)])

    return pl.kernel(
        body,
        out_type=jax.ShapeDtypeStruct((NC * npad,), jnp.float32),
        mesh=mesh,
        scratch_types=[
            pltpu.VMEM((nb, PB), jnp.int32),
            pltpu.VMEM((PB,), jnp.float32),
            pltpu.VMEM((rpt,), jnp.float32),
            pltpu.VMEM_SHARED((npad,), jnp.float32),
        ],
    )


def _scatter_kernel(npad, nb, d):
    mesh = plsc.VectorSubcoreMesh(core_axis_name="c", subcore_axis_name="s")
    rpt = npad // NS
    nb2 = nb // 2

    def body(rows_hbm, cols_hbm, y_hbm, zinit_hbm, out_hbm,
             idxr_v, idxc_v, msg_v, acc_sh, sem):
        c = lax.axis_index("c")
        s = lax.axis_index("s")
        wid = c * NS + s
        base_blk = wid * nb
        pltpu.sync_copy(zinit_hbm.at[pl.ds(s * rpt, rpt)],
                        acc_sh.at[pl.ds(s * rpt, rpt)])
        plsc.subcore_barrier()
        for b in range(2):
            pltpu.sync_copy(rows_hbm.at[base_blk + b], idxr_v.at[b])
            pltpu.sync_copy(cols_hbm.at[base_blk + b], idxc_v.at[b])
            pltpu.async_copy(y_hbm.at[idxr_v.at[b]], msg_v.at[b], sem)

        def pair(k, carry):
            for b in range(2):
                j = 2 * k + b
                pltpu.make_async_copy(y_hbm.at[idxr_v.at[b]], msg_v.at[b],
                                      sem).wait()
                pltpu.sync_copy(msg_v.at[b], acc_sh.at[idxc_v.at[b]],
                                add=True)

                @pl.when(k < nb2 - 1)
                def _():
                    pltpu.sync_copy(rows_hbm.at[base_blk + j + 2],
                                    idxr_v.at[b])
                    pltpu.sync_copy(cols_hbm.at[base_blk + j + 2],
                                    idxc_v.at[b])
                    pltpu.async_copy(y_hbm.at[idxr_v.at[b]], msg_v.at[b], sem)
            return carry

        lax.fori_loop(0, nb2, pair, 0)
        plsc.subcore_barrier()
        pltpu.sync_copy(acc_sh.at[pl.ds(s * rpt, rpt)],
                        out_hbm.at[pl.ds(c * npad + s * rpt, rpt)])

    return pl.kernel(
        body,
        out_type=jax.ShapeDtypeStruct((NC * npad, d), jnp.float32),
        mesh=mesh,
        scratch_types=[
            pltpu.VMEM((2, PB), jnp.int32),
            pltpu.VMEM((2, PB), jnp.int32),
            pltpu.VMEM((2, PB, d), jnp.float32),
            pltpu.VMEM_SHARED((npad, d), jnp.float32),
            pltpu.SemaphoreType.DMA,
        ],
    )


def _transform_kernel(npad, d_in, d_out, br):
    def body(x_ref, wt_ref, d0_ref, d1_ref, y_ref):
        deg = d0_ref[...] + d1_ref[...] + 1.0
        dinv = lax.rsqrt(deg)
        xw = jnp.dot(x_ref[...], wt_ref[...],
                     preferred_element_type=jnp.float32)
        y_ref[...] = xw * dinv

    return pl.pallas_call(
        body,
        grid=(npad // br,),
        in_specs=[
            pl.BlockSpec((br, d_in), lambda i: (i, 0)),
            pl.BlockSpec((d_in, d_out), lambda i: (0, 0)),
            pl.BlockSpec((br, 1), lambda i: (i, 0)),
            pl.BlockSpec((br, 1), lambda i: (i, 0)),
        ],
        out_specs=pl.BlockSpec((br, d_out), lambda i: (i, 0)),
        out_shape=jax.ShapeDtypeStruct((npad, d_out), jnp.float32),
    )


def _final_kernel(n, d, br):
    def body(p0_ref, p1_ref, y_ref, d0_ref, d1_ref, o_ref):
        deg = d0_ref[...] + d1_ref[...] + 1.0
        dinv = lax.rsqrt(deg)
        o_ref[...] = (p0_ref[...] + p1_ref[...] + y_ref[...]) * dinv

    return pl.pallas_call(
        body,
        grid=(n // br,),
        in_specs=[
            pl.BlockSpec((br, d), lambda i: (i, 0)),
            pl.BlockSpec((br, d), lambda i: (i, 0)),
            pl.BlockSpec((br, d), lambda i: (i, 0)),
            pl.BlockSpec((br, 1), lambda i: (i, 0)),
            pl.BlockSpec((br, 1), lambda i: (i, 0)),
        ],
        out_specs=pl.BlockSpec((br, d), lambda i: (i, 0)),
        out_shape=jax.ShapeDtypeStruct((n, d), jnp.float32),
    )


def kernel(x, edge_index, num_nodes, W):
    n, d_in = x.shape
    d_out = W.shape[0]
    e = edge_index.shape[1]
    del num_nodes

    npad = -(-n // (NS * 16)) * (NS * 16)
    if npad == n:
        npad += NS * 16
    nb = -(-e // (NW * PB))
    nb += nb % 2
    ep = NW * nb * PB

    row = edge_index[0]
    col = edge_index[1]
    padn = ep - e
    pad_iota = jnp.arange(padn, dtype=edge_index.dtype)
    rows2 = jnp.concatenate(
        [row, pad_iota % npad]).reshape(NW * nb, PB)
    cols2 = jnp.concatenate(
        [col, n + pad_iota % (npad - n)]).reshape(NW * nb, PB)
    xp = jnp.concatenate([x, jnp.zeros((npad - n, d_in), x.dtype)])
    wt = W.T
    zinit = jnp.zeros((npad, d_out), jnp.float32)

    degp = _deg_kernel(npad, nb)(cols2)
    d0c = degp[:npad].reshape(npad, 1)
    d1c = degp[npad:].reshape(npad, 1)

    y = _transform_kernel(npad, d_in, d_out, 1024)(xp, wt, d0c, d1c)

    accp = _scatter_kernel(npad, nb, d_out)(rows2, cols2, y, zinit)

    out = _final_kernel(n, d_out, 1000)(
        accp[:n], accp[npad:npad + n], y[:n], d0c[:n], d1c[:n])
    return out


# speedup vs baseline: 33.6635x; 1.0092x over previous
import jax
import jax.numpy as jnp
from jax import lax
from jax.experimental import pallas as pl
from jax.experimental.pallas import tpu as pltpu
from jax.experimental.pallas import tpu_sc as plsc

NC = 2
NS = 16
NW = NC * NS
PB = 128


def _deg_kernel(npad, nb):
    mesh = plsc.VectorSubcoreMesh(core_axis_name="c", subcore_axis_name="s")
    rpt = npad // NS

    def body(cols_hbm, deg_out, idx_v, ones_v, zero_v, deg_sh):
        c = lax.axis_index("c")
        s = lax.axis_index("s")
        wid = c * NS + s
        for i in range(PB // 16):
            ones_v[pl.ds(i * 16, 16)] = jnp.ones((16,), jnp.float32)
        for i in range(rpt // 16):
            zero_v[pl.ds(i * 16, 16)] = jnp.zeros((16,), jnp.float32)
        pltpu.sync_copy(zero_v, deg_sh.at[pl.ds(s * rpt, rpt)])
        pltpu.sync_copy(cols_hbm.at[pl.ds(wid * nb, nb)], idx_v)
        plsc.subcore_barrier()

        def blk(j, carry):
            pltpu.sync_copy(ones_v, deg_sh.at[idx_v.at[j]], add=True)
            return carry

        lax.fori_loop(0, nb, blk, 0)
        plsc.subcore_barrier()
        pltpu.sync_copy(deg_sh.at[pl.ds(s * rpt, rpt)],
                        deg_out.at[pl.ds(c * npad + s * rpt, rpt)])

    return pl.kernel(
        body,
        out_type=jax.ShapeDtypeStruct((NC * npad,), jnp.float32),
        mesh=mesh,
        scratch_types=[
            pltpu.VMEM((nb, PB), jnp.int32),
            pltpu.VMEM((PB,), jnp.float32),
            pltpu.VMEM((rpt,), jnp.float32),
            pltpu.VMEM_SHARED((npad,), jnp.float32),
        ],
    )


def _scatter_kernel(npad, nb, d):
    mesh = plsc.VectorSubcoreMesh(core_axis_name="c", subcore_axis_name="s")
    rpt = npad // NS
    nb2 = nb // 2

    def body(rows_hbm, cols_hbm, y_hbm, out_hbm,
             idxr_v, idxc_v, msg_v, acc_sh, sem):
        c = lax.axis_index("c")
        s = lax.axis_index("s")
        wid = c * NS + s
        base_blk = wid * nb
        pltpu.sync_copy(y_hbm.at[pl.ds(s * rpt, rpt)],
                        acc_sh.at[pl.ds(s * rpt, rpt)])
        plsc.subcore_barrier()
        for b in range(2):
            pltpu.sync_copy(rows_hbm.at[base_blk + b], idxr_v.at[b])
            pltpu.sync_copy(cols_hbm.at[base_blk + b], idxc_v.at[b])
            pltpu.async_copy(y_hbm.at[idxr_v.at[b]], msg_v.at[b], sem)

        def pair(k, carry):
            for b in range(2):
                j = 2 * k + b
                pltpu.make_async_copy(y_hbm.at[idxr_v.at[b]], msg_v.at[b],
                                      sem).wait()
                pltpu.sync_copy(msg_v.at[b], acc_sh.at[idxc_v.at[b]],
                                add=True)

                @pl.when(k < nb2 - 1)
                def _():
                    pltpu.sync_copy(rows_hbm.at[base_blk + j + 2],
                                    idxr_v.at[b])
                    pltpu.sync_copy(cols_hbm.at[base_blk + j + 2],
                                    idxc_v.at[b])
                    pltpu.async_copy(y_hbm.at[idxr_v.at[b]], msg_v.at[b], sem)
            return carry

        lax.fori_loop(0, nb2, pair, 0)
        plsc.subcore_barrier()
        pltpu.sync_copy(acc_sh.at[pl.ds(s * rpt, rpt)],
                        out_hbm.at[pl.ds(c * npad + s * rpt, rpt)])

    return pl.kernel(
        body,
        out_type=jax.ShapeDtypeStruct((NC * npad, d), jnp.float32),
        mesh=mesh,
        scratch_types=[
            pltpu.VMEM((2, PB), jnp.int32),
            pltpu.VMEM((2, PB), jnp.int32),
            pltpu.VMEM((2, PB, d), jnp.float32),
            pltpu.VMEM_SHARED((npad, d), jnp.float32),
            pltpu.SemaphoreType.DMA,
        ],
    )


def _transform_kernel(n, npad, d_in, d_out, br):
    nblk = npad // br

    def body(x_ref, wt_ref, d0_ref, d1_ref, y_ref):
        deg = d0_ref[...] + d1_ref[...] + 1.0
        dinv = lax.rsqrt(deg)
        xw = jnp.dot(x_ref[...], wt_ref[...],
                     preferred_element_type=jnp.float32)
        y_ref[...] = xw * dinv

    return pl.pallas_call(
        body,
        grid=(nblk,),
        in_specs=[
            pl.BlockSpec((br, d_in), lambda i: (i, 0)),
            pl.BlockSpec((d_in, d_out), lambda i: (0, 0)),
            pl.BlockSpec((br, 1), lambda i: (i, 0)),
            pl.BlockSpec((br, 1), lambda i: (nblk + i, 0)),
        ],
        out_specs=pl.BlockSpec((br, d_out), lambda i: (i, 0)),
        out_shape=jax.ShapeDtypeStruct((npad, d_out), jnp.float32),
    )


def _final_kernel(n, npad, d, br):
    nblk = npad // br

    def body(acc0_ref, acc1_ref, y_ref, d0_ref, d1_ref, o_ref):
        deg = d0_ref[...] + d1_ref[...] + 1.0
        dinv = lax.rsqrt(deg)
        o_ref[...] = (acc0_ref[...] + acc1_ref[...] - y_ref[...]) * dinv

    return pl.pallas_call(
        body,
        grid=(nblk,),
        in_specs=[
            pl.BlockSpec((br, d), lambda i: (i, 0)),
            pl.BlockSpec((br, d), lambda i: (nblk + i, 0)),
            pl.BlockSpec((br, d), lambda i: (i, 0)),
            pl.BlockSpec((br, 1), lambda i: (i, 0)),
            pl.BlockSpec((br, 1), lambda i: (nblk + i, 0)),
        ],
        out_specs=pl.BlockSpec((br, d), lambda i: (i, 0)),
        out_shape=jax.ShapeDtypeStruct((n, d), jnp.float32),
    )


def kernel(x, edge_index, num_nodes, W):
    n, d_in = x.shape
    d_out = W.shape[0]
    e = edge_index.shape[1]
    del num_nodes

    npad = -(-n // (NS * 16)) * (NS * 16)
    if npad == n:
        npad += NS * 16
    nb = -(-e // (NW * PB))
    nb += nb % 2
    ep = NW * nb * PB

    row = edge_index[0]
    col = edge_index[1]
    padn = ep - e
    pad_iota = jnp.arange(padn, dtype=edge_index.dtype)
    rows2 = jnp.concatenate(
        [row, pad_iota % n]).reshape(NW * nb, PB)
    cols2 = jnp.concatenate(
        [col, n + pad_iota % (npad - n)]).reshape(NW * nb, PB)
    wt = W.T

    degp = _deg_kernel(npad, nb)(cols2)
    dcol = degp.reshape(NC * npad, 1)

    y = _transform_kernel(n, npad, d_in, d_out, 512)(x, wt, dcol, dcol)

    accp = _scatter_kernel(npad, nb, d_out)(rows2, cols2, y)

    out = _final_kernel(n, npad, d_out, 512)(accp, accp, y, dcol, dcol)
    return out
